# baseline (speedup 1.0000x reference)
import functools

import jax
import jax.numpy as jnp
from jax import lax
from jax.experimental import pallas as pl
from jax.experimental.pallas import tpu as pltpu
from jax.experimental.pallas import tpu_sc as plsc

_NC = 2
_NS = 16
_NW = _NC * _NS


@functools.lru_cache(maxsize=None)
def _gather1_kernel(b_flat: int, v: int, d: int, phase: int):
    b_per_w = b_flat // _NW
    mesh = plsc.VectorSubcoreMesh(core_axis_name="c", subcore_axis_name="s")

    @functools.partial(
        pl.kernel,
        mesh=mesh,
        out_type=jax.ShapeDtypeStruct((b_flat, d), jnp.float32),
        scratch_types=[
            pltpu.VMEM((b_per_w,), jnp.int32),
            pltpu.VMEM((b_per_w, d), jnp.float32),
            pltpu.SemaphoreType.DMA,
        ],
        compiler_params=pltpu.CompilerParams(
            use_tc_tiling_on_sc=False, needs_layout_passes=False
        ),
    )
    def k(idx_hbm, tab_hbm, out_hbm, raw_v, rows_v, sem):
        wid = lax.axis_index("s") * _NC + lax.axis_index("c")
        base = wid * b_per_w
        pltpu.sync_copy(idx_hbm.at[pl.ds(phase * b_flat + base, b_per_w)],
                        raw_v)
        pltpu.async_copy(tab_hbm.at[raw_v], rows_v, sem).wait()
        pltpu.sync_copy(rows_v, out_hbm.at[pl.ds(base, b_per_w)])

    return k


def kernel(dx_ints, proc_ints, dx_table, proc_table, visit_param, max_num_codes):
    b, l = dx_ints.shape
    v = dx_table.shape[0]
    d = dx_table.shape[1]
    b_flat = b * l
    idx_cat = jnp.concatenate(
        [dx_ints.reshape(b_flat), proc_ints.reshape(b_flat)]
    )
    emb_dx_f = _gather1_kernel(b_flat, v, d, 0)(idx_cat, dx_table)
    emb_proc_f = _gather1_kernel(b_flat, v, d, 1)(idx_cat, proc_table)
    emb_dx = emb_dx_f.reshape(b, l, d)
    emb_proc = emb_proc_f.reshape(b, l, d)
    mask_dx = jnp.ones((b, l, 1), dtype=jnp.float32)
    mask_proc = jnp.ones((b, l, 1), dtype=jnp.float32)
    visit_emb = jnp.broadcast_to(visit_param[None, :, :], (1, 1, d))
    mask_visit = jnp.ones((1, 1), dtype=jnp.float32)
    return (emb_dx, emb_proc, visit_emb, mask_dx, mask_proc, mask_visit)


# speedup vs baseline: 1.3098x; 1.3098x over previous
import functools

import jax
import jax.numpy as jnp
from jax import lax
from jax.experimental import pallas as pl
from jax.experimental.pallas import tpu as pltpu
from jax.experimental.pallas import tpu_sc as plsc

_NC = 2
_NS = 16
_NW = _NC * _NS


@functools.lru_cache(maxsize=None)
def _gather1_kernel(b_flat: int, v: int, d: int, phase: int):
    b_per_w = b_flat // _NW
    sub = b_per_w // 8
    out_rows = b_flat * d // 128
    orow_per_w = b_per_w * d // 128
    mesh = plsc.VectorSubcoreMesh(core_axis_name="c", subcore_axis_name="s")

    @functools.partial(
        pl.kernel,
        mesh=mesh,
        out_type=jax.ShapeDtypeStruct((out_rows, 128), jnp.float32),
        scratch_types=[
            pltpu.VMEM((b_per_w,), jnp.int32),
            pltpu.VMEM((8, sub), jnp.int32),
            pltpu.VMEM((8, sub, d), jnp.float32),
            pltpu.SemaphoreType.DMA,
            pltpu.SemaphoreType.DMA,
        ],
        compiler_params=pltpu.CompilerParams(
            use_tc_tiling_on_sc=False, needs_layout_passes=False
        ),
    )
    def k(idx_hbm, tab_hbm, out_hbm, raw_v, idx_v, rows_v, gsem, osem):
        wid = lax.axis_index("s") * _NC + lax.axis_index("c")
        base = wid * b_per_w
        obase = wid * orow_per_w
        lane = lax.iota(jnp.int32, 16)

        pltpu.sync_copy(idx_hbm.at[pl.ds(phase * b_flat + base, b_per_w)],
                        raw_v)

        def reorder(g, _):
            offs = g * 128 + lane * 8
            for s in range(8):
                idx_v[s, pl.ds(g * 16, 16)] = plsc.load_gather(
                    raw_v, [offs + s]
                )
            return 0

        lax.fori_loop(0, sub // 16, reorder, 0)

        for s in range(8):
            pltpu.make_async_copy(
                tab_hbm.at[idx_v.at[s]], rows_v.at[s], gsem
            ).start()
        for s in range(8):
            pltpu.make_async_copy(
                tab_hbm.at[idx_v.at[s]], rows_v.at[s], gsem
            ).wait()
        for s in range(8):
            pltpu.make_async_copy(
                rows_v.at[s],
                out_hbm.at[pl.ds(obase, orow_per_w), pl.ds(d * s, d)],
                osem,
            ).start()
        for s in range(8):
            pltpu.make_async_copy(
                rows_v.at[s],
                out_hbm.at[pl.ds(obase, orow_per_w), pl.ds(d * s, d)],
                osem,
            ).wait()

    return k


def kernel(dx_ints, proc_ints, dx_table, proc_table, visit_param, max_num_codes):
    b, l = dx_ints.shape
    v = dx_table.shape[0]
    d = dx_table.shape[1]
    b_flat = b * l
    idx_cat = jnp.concatenate(
        [dx_ints.reshape(b_flat), proc_ints.reshape(b_flat)]
    )
    emb_dx128 = _gather1_kernel(b_flat, v, d, 0)(idx_cat, dx_table)
    emb_proc128 = _gather1_kernel(b_flat, v, d, 1)(idx_cat, proc_table)
    emb_dx = emb_dx128.reshape(b, l, d)
    emb_proc = emb_proc128.reshape(b, l, d)
    mask_dx = jnp.ones((b, l, 1), dtype=jnp.float32)
    mask_proc = jnp.ones((b, l, 1), dtype=jnp.float32)
    visit_emb = jnp.broadcast_to(visit_param[None, :, :], (1, 1, d))
    mask_visit = jnp.ones((1, 1), dtype=jnp.float32)
    return (emb_dx, emb_proc, visit_emb, mask_dx, mask_proc, mask_visit)
